# Initial kernel scaffold; baseline (speedup 1.0000x reference)
import jax
import jax.numpy as jnp
from jax.experimental import pallas as pl


def kernel(x_dict, edge_index_dict, edge_attr_dict, W_src, W_dst, att_src, att_dst, bias):
    raise NotImplementedError("write your pallas kernel here")


# speedup vs baseline: 19.7143x; 19.7143x over previous
import jax
import jax.numpy as jnp
from jax import lax
from jax.experimental import pallas as pl
from jax.experimental.pallas import tpu as pltpu
from jax.experimental.pallas import tpu_sc as plsc

N = 10000
D = 128
H = 2
C = 256
HC = H * C

NP = 10240
RPT = NP // 16
BN = 512
CH = 64
G = 8
NG = 41
NCH = G * NG
E2P = 16 * CH * NCH


def _tc_body(x_ref, ws_ref, wd_ref, as_ref, ad_ref, h4_ref, a4_ref):
    xb = x_ref[...]
    hs = jnp.dot(xb, ws_ref[...], preferred_element_type=jnp.float32)
    hd = jnp.dot(xb, wd_ref[...], preferred_element_type=jnp.float32)
    asv = as_ref[...].reshape(H, C)
    adv = ad_ref[...].reshape(H, C)
    cols = []
    for h in range(H):
        cols.append(jnp.sum(hs[:, h * C:(h + 1) * C] * asv[h][None, :],
                            axis=1, keepdims=True))
    for h in range(H):
        cols.append(jnp.sum(hd[:, h * C:(h + 1) * C] * adv[h][None, :],
                            axis=1, keepdims=True))
    a4_ref[...] = jnp.concatenate(cols, axis=1)
    for s in range(4):
        h4_ref[s] = hs[:, s * 128:(s + 1) * 128]


def _tc_call(xp, W_src, W_dst, att_src, att_dst):
    return pl.pallas_call(
        _tc_body,
        grid=(NP // BN,),
        in_specs=[
            pl.BlockSpec((BN, D), lambda i: (i, 0)),
            pl.BlockSpec((D, HC), lambda i: (0, 0)),
            pl.BlockSpec((D, HC), lambda i: (0, 0)),
            pl.BlockSpec((1, H, C), lambda i: (0, 0, 0)),
            pl.BlockSpec((1, H, C), lambda i: (0, 0, 0)),
        ],
        out_specs=[
            pl.BlockSpec((4, BN, 128), lambda i: (0, i, 0)),
            pl.BlockSpec((BN, 4), lambda i: (i, 0)),
        ],
        out_shape=[
            jax.ShapeDtypeStruct((4, NP, 128), jnp.float32),
            jax.ShapeDtypeStruct((NP, 4), jnp.float32),
        ],
    )(xp, W_src, W_dst, att_src, att_dst)


def _sc_body(h4_hbm, a4t_hbm, src_hbm, dst_hbm, bias_hbm, out_hbm,
             asrc_v, adst_v, denom_v, src_b, dst_b, rows_v, wbuf_v, idx_v,
             bias_v, tmp_v, dslice_v, sdenoms, sdfinal, acc_s, sem):
    core = lax.axis_index("c")
    sid = lax.axis_index("s")
    zero16 = jnp.zeros((16,), jnp.float32)
    grow = sid * NCH

    pltpu.sync_copy(a4t_hbm.at[core], asrc_v)
    pltpu.sync_copy(a4t_hbm.at[2 + core], adst_v)

    def zero_denom(i, _):
        denom_v[pl.ds(i * 16, 16)] = zero16
        return 0
    lax.fori_loop(0, NP // 16, zero_denom, 0)

    def p1g(g, _):
        pltpu.sync_copy(src_hbm.at[pl.ds(grow + g * G, G)], src_b)
        pltpu.sync_copy(dst_hbm.at[pl.ds(grow + g * G, G)], dst_b)

        def p1(i, _):
            cc = i // 4
            l = i - cc * 4
            s16 = src_b[cc, pl.ds(l * 16, 16)]
            d16 = dst_b[cc, pl.ds(l * 16, 16)]
            al = (plsc.load_gather(asrc_v, [s16])
                  + plsc.load_gather(adst_v, [d16]))
            al = jnp.where(al >= 0, al, 0.2 * al)
            plsc.addupdate_scatter(denom_v, [d16], jnp.exp(al))
            return 0
        lax.fori_loop(0, G * 4, p1, 0)
        return 0
    lax.fori_loop(0, NG, p1g, 0)

    base = sid * RPT

    def zero_dslice(j, _):
        dslice_v[pl.ds(j * 16, 16)] = zero16
        return 0
    lax.fori_loop(0, RPT // 16, zero_dslice, 0)

    for r in range(4):
        @pl.when((sid >= r * 4) & (sid < r * 4 + 4))
        def _():
            pltpu.sync_copy(denom_v, sdenoms.at[sid - r * 4])
        plsc.subcore_barrier()

        def rsum(p, _):
            pltpu.sync_copy(sdenoms.at[p, pl.ds(base, RPT)], tmp_v)

            def rj(j, _):
                sl = pl.ds(j * 16, 16)
                dslice_v[sl] = dslice_v[sl] + tmp_v[sl]
                return 0
            lax.fori_loop(0, RPT // 16, rj, 0)
            return 0
        lax.fori_loop(0, 4, rsum, 0)
        plsc.subcore_barrier()
    pltpu.sync_copy(dslice_v, sdfinal.at[pl.ds(base, RPT)])
    plsc.subcore_barrier()
    pltpu.sync_copy(sdfinal, denom_v)

    for hc in range(2):
        hslot = core * 2 + hc
        rowoff = hslot * NP

        def zero_rows(i, _):
            rows_v[i // 8, pl.ds((i % 8) * 16, 16)] = zero16
            return 0
        lax.fori_loop(0, CH * 8, zero_rows, 0)
        for q in range(RPT // CH):
            pltpu.sync_copy(rows_v, acc_s.at[pl.ds(base + q * CH, CH)])
        pltpu.sync_copy(bias_hbm.at[hslot], bias_v)
        plsc.subcore_barrier()

        def p2g(g, _):
            pltpu.sync_copy(src_hbm.at[pl.ds(grow + g * G, G)], src_b)
            pltpu.sync_copy(dst_hbm.at[pl.ds(grow + g * G, G)], dst_b)

            def p2(cc, _):
                def bi(v, _):
                    sl = pl.ds(v * 16, 16)
                    idx_v[sl] = src_b[cc, sl] + rowoff
                    return 0
                lax.fori_loop(0, 4, bi, 0)
                pltpu.async_copy(h4_hbm.at[idx_v], rows_v, sem).wait()

                def wv(v, _):
                    sl = pl.ds(v * 16, 16)
                    s16 = src_b[cc, sl]
                    d16 = dst_b[cc, sl]
                    al = (plsc.load_gather(asrc_v, [s16])
                          + plsc.load_gather(adst_v, [d16]))
                    al = jnp.where(al >= 0, al, 0.2 * al)
                    den = plsc.load_gather(denom_v, [d16])
                    wbuf_v[sl] = jnp.exp(al) / (den + 1e-16)
                    return 0
                lax.fori_loop(0, 4, wv, 0)

                def sk(k, _):
                    wk = plsc.load_gather(wbuf_v,
                                          [jnp.full((16,), k, jnp.int32)])
                    for j in range(8):
                        sl = pl.ds(j * 16, 16)
                        rows_v[k, sl] = rows_v[k, sl] * wk
                    return 0
                lax.fori_loop(0, CH, sk, 0)

                pltpu.sync_copy(rows_v, acc_s.at[dst_b.at[cc]], add=True)
                return 0
            lax.fori_loop(0, G, p2, 0)
            return 0
        lax.fori_loop(0, NG, p2g, 0)
        plsc.subcore_barrier()

        for q in range(RPT // CH):
            rbase = base + q * CH
            pltpu.sync_copy(acc_s.at[pl.ds(rbase, CH)], rows_v)

            def ab(i, _):
                sl = pl.ds((i % 8) * 16, 16)
                rows_v[i // 8, sl] = rows_v[i // 8, sl] + bias_v[sl]
                return 0
            lax.fori_loop(0, CH * 8, ab, 0)
            pltpu.sync_copy(rows_v, out_hbm.at[hslot, pl.ds(rbase, CH)])
        plsc.subcore_barrier()


_sc_call = pl.kernel(
    _sc_body,
    out_type=jax.ShapeDtypeStruct((4, NP, 128), jnp.float32),
    mesh=plsc.VectorSubcoreMesh(core_axis_name="c", subcore_axis_name="s"),
    compiler_params=pltpu.CompilerParams(needs_layout_passes=False),
    scratch_types=[
        pltpu.VMEM((NP,), jnp.float32),
        pltpu.VMEM((NP,), jnp.float32),
        pltpu.VMEM((NP,), jnp.float32),
        pltpu.VMEM((G, CH), jnp.int32),
        pltpu.VMEM((G, CH), jnp.int32),
        pltpu.VMEM((CH, 128), jnp.float32),
        pltpu.VMEM((CH,), jnp.float32),
        pltpu.VMEM((CH,), jnp.int32),
        pltpu.VMEM((128,), jnp.float32),
        pltpu.VMEM((RPT,), jnp.float32),
        pltpu.VMEM((RPT,), jnp.float32),
        pltpu.VMEM_SHARED((4, NP), jnp.float32),
        pltpu.VMEM_SHARED((NP,), jnp.float32),
        pltpu.VMEM_SHARED((NP, 128), jnp.float32),
        pltpu.SemaphoreType.DMA,
    ],
)


def kernel(x_dict, edge_index_dict, edge_attr_dict, W_src, W_dst, att_src,
           att_dst, bias):
    x = x_dict
    ei = edge_index_dict
    e_in = ei.shape[1]
    e2 = e_in + N
    loop = jnp.arange(N, dtype=jnp.int32)
    src = jnp.concatenate([ei[0].astype(jnp.int32), loop,
                           jnp.zeros((E2P - e2,), jnp.int32)])
    dst = jnp.concatenate([ei[1].astype(jnp.int32), loop,
                           jnp.full((E2P - e2,), N, jnp.int32)])
    src2d = src.reshape(16 * NCH, CH)
    dst2d = dst.reshape(16 * NCH, CH)

    xp = jnp.pad(x, ((0, NP - N), (0, 0)))
    h4, a4 = _tc_call(xp, W_src, W_dst, att_src, att_dst)
    h4flat = h4.reshape(4 * NP, 128)
    a4t = jnp.transpose(a4, (1, 0))
    bias4 = bias.reshape(H, 2, 128).reshape(4, 128)

    out4 = _sc_call(h4flat, a4t, src2d, dst2d, bias4)
    out = out4[:, :N, :]
    out = jnp.transpose(out, (1, 0, 2)).reshape(N, HC)
    return out


# speedup vs baseline: 21.8132x; 1.1065x over previous
import jax
import jax.numpy as jnp
from jax import lax
from jax.experimental import pallas as pl
from jax.experimental.pallas import tpu as pltpu
from jax.experimental.pallas import tpu_sc as plsc

N = 10000
D = 128
H = 2
C = 256
HC = H * C

NP = 10240
RPT = NP // 16
BN = 512
CH = 32
G = 8
NG = 82
NCH = G * NG
E2P = 16 * CH * NCH


def _tc_body(x_ref, ws_ref, wd_ref, as_ref, ad_ref, h4_ref, a4_ref):
    xb = x_ref[...]
    hs = jnp.dot(xb, ws_ref[...], preferred_element_type=jnp.float32)
    hd = jnp.dot(xb, wd_ref[...], preferred_element_type=jnp.float32)
    asv = as_ref[...].reshape(H, C)
    adv = ad_ref[...].reshape(H, C)
    cols = []
    for h in range(H):
        cols.append(jnp.sum(hs[:, h * C:(h + 1) * C] * asv[h][None, :],
                            axis=1, keepdims=True))
    for h in range(H):
        cols.append(jnp.sum(hd[:, h * C:(h + 1) * C] * adv[h][None, :],
                            axis=1, keepdims=True))
    a4_ref[...] = jnp.concatenate(cols, axis=1)
    for s in range(4):
        h4_ref[s] = hs[:, s * 128:(s + 1) * 128]


def _tc_call(xp, W_src, W_dst, att_src, att_dst):
    return pl.pallas_call(
        _tc_body,
        grid=(NP // BN,),
        in_specs=[
            pl.BlockSpec((BN, D), lambda i: (i, 0)),
            pl.BlockSpec((D, HC), lambda i: (0, 0)),
            pl.BlockSpec((D, HC), lambda i: (0, 0)),
            pl.BlockSpec((1, H, C), lambda i: (0, 0, 0)),
            pl.BlockSpec((1, H, C), lambda i: (0, 0, 0)),
        ],
        out_specs=[
            pl.BlockSpec((4, BN, 128), lambda i: (0, i, 0)),
            pl.BlockSpec((BN, 4), lambda i: (i, 0)),
        ],
        out_shape=[
            jax.ShapeDtypeStruct((4, NP, 128), jnp.float32),
            jax.ShapeDtypeStruct((NP, 4), jnp.float32),
        ],
    )(xp, W_src, W_dst, att_src, att_dst)


def _sc_body(h4_hbm, a4t_hbm, src_hbm, dst_hbm, bias_hbm, out_hbm,
             asrc_v, adst_v, denom_v, src_b, dst_b, rows0_v, rows1_v,
             wbuf_v, idx0_v, idx1_v, bias_v, tmp_v, dslice_v,
             sdenoms, sdfinal, acc_s, semg0, semg1, sems0, sems1):
    core = lax.axis_index("c")
    sid = lax.axis_index("s")
    zero16 = jnp.zeros((16,), jnp.float32)
    grow = sid * NCH
    rows_b = (rows0_v, rows1_v)
    idx_b = (idx0_v, idx1_v)
    semg_b = (semg0, semg1)
    sems_b = (sems0, sems1)
    LPC = CH // 16

    pltpu.sync_copy(a4t_hbm.at[core], asrc_v)
    pltpu.sync_copy(a4t_hbm.at[2 + core], adst_v)

    def zero_denom(i, _):
        denom_v[pl.ds(i * 16, 16)] = zero16
        return 0
    lax.fori_loop(0, NP // 16, zero_denom, 0)

    def p1g(g, _):
        pltpu.sync_copy(src_hbm.at[pl.ds(grow + g * G, G)], src_b)
        pltpu.sync_copy(dst_hbm.at[pl.ds(grow + g * G, G)], dst_b)

        def p1(i, _):
            cc = i // LPC
            l = i - cc * LPC
            s16 = src_b[cc, pl.ds(l * 16, 16)]
            d16 = dst_b[cc, pl.ds(l * 16, 16)]
            al = (plsc.load_gather(asrc_v, [s16])
                  + plsc.load_gather(adst_v, [d16]))
            al = jnp.where(al >= 0, al, 0.2 * al)
            plsc.addupdate_scatter(denom_v, [d16], jnp.exp(al))
            return 0
        lax.fori_loop(0, G * LPC, p1, 0)
        return 0
    lax.fori_loop(0, NG, p1g, 0)

    base = sid * RPT

    def zero_dslice(j, _):
        dslice_v[pl.ds(j * 16, 16)] = zero16
        return 0
    lax.fori_loop(0, RPT // 16, zero_dslice, 0)

    for r in range(4):
        @pl.when((sid >= r * 4) & (sid < r * 4 + 4))
        def _():
            pltpu.sync_copy(denom_v, sdenoms.at[sid - r * 4])
        plsc.subcore_barrier()

        def rsum(p, _):
            pltpu.sync_copy(sdenoms.at[p, pl.ds(base, RPT)], tmp_v)

            def rj(j, _):
                sl = pl.ds(j * 16, 16)
                dslice_v[sl] = dslice_v[sl] + tmp_v[sl]
                return 0
            lax.fori_loop(0, RPT // 16, rj, 0)
            return 0
        lax.fori_loop(0, 4, rsum, 0)
        plsc.subcore_barrier()
    pltpu.sync_copy(dslice_v, sdfinal.at[pl.ds(base, RPT)])
    plsc.subcore_barrier()
    pltpu.sync_copy(sdfinal, denom_v)

    for hc in range(2):
        hslot = core * 2 + hc
        rowoff = hslot * NP

        def zero_rows(i, _):
            rows0_v[i // 8, pl.ds((i % 8) * 16, 16)] = zero16
            return 0
        lax.fori_loop(0, CH * 8, zero_rows, 0)
        for q in range(RPT // CH):
            pltpu.sync_copy(rows0_v, acc_s.at[pl.ds(base + q * CH, CH)])
        pltpu.sync_copy(bias_hbm.at[hslot], bias_v)
        plsc.subcore_barrier()

        def fire_gather(cc, buf):
            def bi(v, _):
                sl = pl.ds(v * 16, 16)
                idx_b[buf][sl] = src_b[cc, sl] + rowoff
                return 0
            lax.fori_loop(0, LPC, bi, 0)
            return pltpu.async_copy(h4_hbm.at[idx_b[buf]], rows_b[buf],
                                    semg_b[buf])

        def scale(cc, buf):
            def wv(v, _):
                sl = pl.ds(v * 16, 16)
                s16 = src_b[cc, sl]
                d16 = dst_b[cc, sl]
                al = (plsc.load_gather(asrc_v, [s16])
                      + plsc.load_gather(adst_v, [d16]))
                al = jnp.where(al >= 0, al, 0.2 * al)
                den = plsc.load_gather(denom_v, [d16])
                wbuf_v[sl] = jnp.exp(al) / (den + 1e-16)
                return 0
            lax.fori_loop(0, LPC, wv, 0)

            def sk(k, _):
                wk = plsc.load_gather(wbuf_v, [jnp.full((16,), k, jnp.int32)])
                for j in range(8):
                    sl = pl.ds(j * 16, 16)
                    rows_b[buf][k, sl] = rows_b[buf][k, sl] * wk
                return 0
            lax.fori_loop(0, CH, sk, 0)

        def p2g(g, _):
            pltpu.sync_copy(src_hbm.at[pl.ds(grow + g * G, G)], src_b)
            pltpu.sync_copy(dst_hbm.at[pl.ds(grow + g * G, G)], dst_b)
            gat = fire_gather(0, 0)
            pend = [None, None]
            for cc in range(G):
                buf = cc % 2
                if cc + 1 < G:
                    if pend[1 - buf] is not None:
                        pend[1 - buf].wait()
                    nxt = fire_gather(cc + 1, 1 - buf)
                gat.wait()
                scale(cc, buf)
                pend[buf] = pltpu.async_copy(
                    rows_b[buf], acc_s.at[dst_b.at[cc]], sems_b[buf],
                    add=True)
                if cc + 1 < G:
                    gat = nxt
            pend[0].wait()
            pend[1].wait()
            return 0
        lax.fori_loop(0, NG, p2g, 0)
        plsc.subcore_barrier()

        for q in range(RPT // CH):
            rbase = base + q * CH
            pltpu.sync_copy(acc_s.at[pl.ds(rbase, CH)], rows0_v)

            def ab(i, _):
                sl = pl.ds((i % 8) * 16, 16)
                rows0_v[i // 8, sl] = rows0_v[i // 8, sl] + bias_v[sl]
                return 0
            lax.fori_loop(0, CH * 8, ab, 0)
            pltpu.sync_copy(rows0_v, out_hbm.at[hslot, pl.ds(rbase, CH)])
        plsc.subcore_barrier()


_sc_call = pl.kernel(
    _sc_body,
    out_type=jax.ShapeDtypeStruct((4, NP, 128), jnp.float32),
    mesh=plsc.VectorSubcoreMesh(core_axis_name="c", subcore_axis_name="s"),
    compiler_params=pltpu.CompilerParams(needs_layout_passes=False),
    scratch_types=[
        pltpu.VMEM((NP,), jnp.float32),
        pltpu.VMEM((NP,), jnp.float32),
        pltpu.VMEM((NP,), jnp.float32),
        pltpu.VMEM((G, CH), jnp.int32),
        pltpu.VMEM((G, CH), jnp.int32),
        pltpu.VMEM((CH, 128), jnp.float32),
        pltpu.VMEM((CH, 128), jnp.float32),
        pltpu.VMEM((CH,), jnp.float32),
        pltpu.VMEM((CH,), jnp.int32),
        pltpu.VMEM((CH,), jnp.int32),
        pltpu.VMEM((128,), jnp.float32),
        pltpu.VMEM((RPT,), jnp.float32),
        pltpu.VMEM((RPT,), jnp.float32),
        pltpu.VMEM_SHARED((4, NP), jnp.float32),
        pltpu.VMEM_SHARED((NP,), jnp.float32),
        pltpu.VMEM_SHARED((NP, 128), jnp.float32),
        pltpu.SemaphoreType.DMA,
        pltpu.SemaphoreType.DMA,
        pltpu.SemaphoreType.DMA,
        pltpu.SemaphoreType.DMA,
    ],
)


def kernel(x_dict, edge_index_dict, edge_attr_dict, W_src, W_dst, att_src,
           att_dst, bias):
    x = x_dict
    ei = edge_index_dict
    e_in = ei.shape[1]
    e2 = e_in + N
    loop = jnp.arange(N, dtype=jnp.int32)
    src = jnp.concatenate([ei[0].astype(jnp.int32), loop,
                           jnp.zeros((E2P - e2,), jnp.int32)])
    dst = jnp.concatenate([ei[1].astype(jnp.int32), loop,
                           jnp.full((E2P - e2,), N, jnp.int32)])
    src2d = src.reshape(16 * NCH, CH)
    dst2d = dst.reshape(16 * NCH, CH)

    xp = jnp.pad(x, ((0, NP - N), (0, 0)))
    h4, a4 = _tc_call(xp, W_src, W_dst, att_src, att_dst)
    h4flat = h4.reshape(4 * NP, 128)
    a4t = jnp.transpose(a4, (1, 0))
    bias4 = bias.reshape(H, 2, 128).reshape(4, 128)

    out4 = _sc_call(h4flat, a4t, src2d, dst2d, bias4)
    out = out4[:, :N, :]
    out = jnp.transpose(out, (1, 0, 2)).reshape(N, HC)
    return out


# speedup vs baseline: 22.8582x; 1.0479x over previous
import jax
import jax.numpy as jnp
from jax import lax
from jax.experimental import pallas as pl
from jax.experimental.pallas import tpu as pltpu
from jax.experimental.pallas import tpu_sc as plsc

N = 10000
D = 128
H = 2
C = 256
HC = H * C

NP = 10240
RPT = NP // 16
BN = 512
CH = 32
G = 8
NG = 82
NCH = G * NG
E2P = 16 * CH * NCH


def _tc_body(x_ref, ws_ref, wd_ref, as_ref, ad_ref, h4_ref, a4_ref):
    xb = x_ref[...]
    hs = jnp.dot(xb, ws_ref[...], preferred_element_type=jnp.float32)
    hd = jnp.dot(xb, wd_ref[...], preferred_element_type=jnp.float32)
    asv = as_ref[...].reshape(H, C)
    adv = ad_ref[...].reshape(H, C)
    cols = []
    for h in range(H):
        cols.append(jnp.sum(hs[:, h * C:(h + 1) * C] * asv[h][None, :],
                            axis=1, keepdims=True))
    for h in range(H):
        cols.append(jnp.sum(hd[:, h * C:(h + 1) * C] * adv[h][None, :],
                            axis=1, keepdims=True))
    a4_ref[...] = jnp.concatenate(cols, axis=1)
    for s in range(4):
        h4_ref[s] = hs[:, s * 128:(s + 1) * 128]


def _tc_call(xp, W_src, W_dst, att_src, att_dst):
    return pl.pallas_call(
        _tc_body,
        grid=(NP // BN,),
        in_specs=[
            pl.BlockSpec((BN, D), lambda i: (i, 0)),
            pl.BlockSpec((D, HC), lambda i: (0, 0)),
            pl.BlockSpec((D, HC), lambda i: (0, 0)),
            pl.BlockSpec((1, H, C), lambda i: (0, 0, 0)),
            pl.BlockSpec((1, H, C), lambda i: (0, 0, 0)),
        ],
        out_specs=[
            pl.BlockSpec((4, BN, 128), lambda i: (0, i, 0)),
            pl.BlockSpec((BN, 4), lambda i: (i, 0)),
        ],
        out_shape=[
            jax.ShapeDtypeStruct((4, NP, 128), jnp.float32),
            jax.ShapeDtypeStruct((NP, 4), jnp.float32),
        ],
    )(xp, W_src, W_dst, att_src, att_dst)


def _sc_body(h4_hbm, a4t_hbm, src_hbm, dst_hbm, bias_hbm, out_hbm,
             asrc_v, adst_v, denom_v, src_b, dst_b, rows0_v, rows1_v,
             rows2_v, wbuf_v, idx0_v, idx1_v, idx2_v, bias_v, tmp_v,
             dslice_v, sdenoms, sdfinal, acc_s, semg0, semg1, semg2,
             sems0, sems1, sems2):
    core = lax.axis_index("c")
    sid = lax.axis_index("s")
    zero16 = jnp.zeros((16,), jnp.float32)
    grow = sid * NCH
    rows_b = (rows0_v, rows1_v, rows2_v)
    idx_b = (idx0_v, idx1_v, idx2_v)
    semg_b = (semg0, semg1, semg2)
    sems_b = (sems0, sems1, sems2)
    LPC = CH // 16

    pltpu.sync_copy(a4t_hbm.at[core], asrc_v)
    pltpu.sync_copy(a4t_hbm.at[2 + core], adst_v)

    def zero_denom(i, _):
        denom_v[pl.ds(i * 16, 16)] = zero16
        return 0
    lax.fori_loop(0, NP // 16, zero_denom, 0)

    def p1g(g, _):
        pltpu.sync_copy(src_hbm.at[pl.ds(grow + g * G, G)], src_b)
        pltpu.sync_copy(dst_hbm.at[pl.ds(grow + g * G, G)], dst_b)

        def p1(i, _):
            cc = i // LPC
            l = i - cc * LPC
            s16 = src_b[cc, pl.ds(l * 16, 16)]
            d16 = dst_b[cc, pl.ds(l * 16, 16)]
            al = (plsc.load_gather(asrc_v, [s16])
                  + plsc.load_gather(adst_v, [d16]))
            al = jnp.where(al >= 0, al, 0.2 * al)
            plsc.addupdate_scatter(denom_v, [d16], jnp.exp(al))
            return 0
        lax.fori_loop(0, G * LPC, p1, 0)
        return 0
    lax.fori_loop(0, NG, p1g, 0)

    base = sid * RPT

    def zero_dslice(j, _):
        dslice_v[pl.ds(j * 16, 16)] = zero16
        return 0
    lax.fori_loop(0, RPT // 16, zero_dslice, 0)

    for r in range(8):
        @pl.when((sid >= r * 2) & (sid < r * 2 + 2))
        def _():
            pltpu.sync_copy(denom_v, sdenoms.at[sid - r * 2])
        plsc.subcore_barrier()

        def rsum(p, _):
            pltpu.sync_copy(sdenoms.at[p, pl.ds(base, RPT)], tmp_v)

            def rj(j, _):
                sl = pl.ds(j * 16, 16)
                dslice_v[sl] = dslice_v[sl] + tmp_v[sl]
                return 0
            lax.fori_loop(0, RPT // 16, rj, 0)
            return 0
        lax.fori_loop(0, 2, rsum, 0)
        plsc.subcore_barrier()
    pltpu.sync_copy(dslice_v, sdfinal.at[pl.ds(base, RPT)])
    plsc.subcore_barrier()
    pltpu.sync_copy(sdfinal, denom_v)

    for hc in range(2):
        hslot = core * 2 + hc
        rowoff = hslot * NP

        def zero_rows(i, _):
            rows0_v[i // 8, pl.ds((i % 8) * 16, 16)] = zero16
            return 0
        lax.fori_loop(0, CH * 8, zero_rows, 0)
        for q in range(RPT // CH):
            pltpu.sync_copy(rows0_v, acc_s.at[pl.ds(base + q * CH, CH)])
        pltpu.sync_copy(bias_hbm.at[hslot], bias_v)
        plsc.subcore_barrier()

        def fire_gather(cc, buf):
            def bi(v, _):
                sl = pl.ds(v * 16, 16)
                idx_b[buf][sl] = src_b[cc, sl] + rowoff
                return 0
            lax.fori_loop(0, LPC, bi, 0)
            return pltpu.async_copy(h4_hbm.at[idx_b[buf]], rows_b[buf],
                                    semg_b[buf])

        def scale(cc, buf):
            def wv(v, _):
                sl = pl.ds(v * 16, 16)
                s16 = src_b[cc, sl]
                d16 = dst_b[cc, sl]
                al = (plsc.load_gather(asrc_v, [s16])
                      + plsc.load_gather(adst_v, [d16]))
                al = jnp.where(al >= 0, al, 0.2 * al)
                den = plsc.load_gather(denom_v, [d16])
                wbuf_v[sl] = jnp.exp(al) / (den + 1e-16)
                return 0
            lax.fori_loop(0, LPC, wv, 0)

            def sk(k, _):
                wk = plsc.load_gather(wbuf_v, [jnp.full((16,), k, jnp.int32)])
                for j in range(8):
                    sl = pl.ds(j * 16, 16)
                    rows_b[buf][k, sl] = rows_b[buf][k, sl] * wk
                return 0
            lax.fori_loop(0, CH, sk, 0)

        def p2g(g, _):
            pltpu.sync_copy(src_hbm.at[pl.ds(grow + g * G, G)], src_b)
            pltpu.sync_copy(dst_hbm.at[pl.ds(grow + g * G, G)], dst_b)
            gat = [None, None, None]
            pend = [None, None, None]
            gat[0] = fire_gather(0, 0)
            gat[1] = fire_gather(1, 1)
            for cc in range(G):
                buf = cc % 3
                if cc + 2 < G:
                    nb = (cc + 2) % 3
                    if pend[nb] is not None:
                        pend[nb].wait()
                    gat[nb] = fire_gather(cc + 2, nb)
                gat[buf].wait()
                scale(cc, buf)
                pend[buf] = pltpu.async_copy(
                    rows_b[buf], acc_s.at[dst_b.at[cc]], sems_b[buf],
                    add=True)
            for b in range(3):
                pend[b].wait()
            return 0
        lax.fori_loop(0, NG, p2g, 0)
        plsc.subcore_barrier()

        for q in range(RPT // CH):
            rbase = base + q * CH
            pltpu.sync_copy(acc_s.at[pl.ds(rbase, CH)], rows0_v)

            def ab(i, _):
                sl = pl.ds((i % 8) * 16, 16)
                rows0_v[i // 8, sl] = rows0_v[i // 8, sl] + bias_v[sl]
                return 0
            lax.fori_loop(0, CH * 8, ab, 0)
            pltpu.sync_copy(rows0_v, out_hbm.at[hslot, pl.ds(rbase, CH)])
        plsc.subcore_barrier()


_sc_call = pl.kernel(
    _sc_body,
    out_type=jax.ShapeDtypeStruct((4, NP, 128), jnp.float32),
    mesh=plsc.VectorSubcoreMesh(core_axis_name="c", subcore_axis_name="s"),
    compiler_params=pltpu.CompilerParams(needs_layout_passes=False),
    scratch_types=[
        pltpu.VMEM((NP,), jnp.float32),
        pltpu.VMEM((NP,), jnp.float32),
        pltpu.VMEM((NP,), jnp.float32),
        pltpu.VMEM((G, CH), jnp.int32),
        pltpu.VMEM((G, CH), jnp.int32),
        pltpu.VMEM((CH, 128), jnp.float32),
        pltpu.VMEM((CH, 128), jnp.float32),
        pltpu.VMEM((CH, 128), jnp.float32),
        pltpu.VMEM((CH,), jnp.float32),
        pltpu.VMEM((CH,), jnp.int32),
        pltpu.VMEM((CH,), jnp.int32),
        pltpu.VMEM((CH,), jnp.int32),
        pltpu.VMEM((128,), jnp.float32),
        pltpu.VMEM((RPT,), jnp.float32),
        pltpu.VMEM((RPT,), jnp.float32),
        pltpu.VMEM_SHARED((2, NP), jnp.float32),
        pltpu.VMEM_SHARED((NP,), jnp.float32),
        pltpu.VMEM_SHARED((NP, 128), jnp.float32),
        pltpu.SemaphoreType.DMA,
        pltpu.SemaphoreType.DMA,
        pltpu.SemaphoreType.DMA,
        pltpu.SemaphoreType.DMA,
        pltpu.SemaphoreType.DMA,
        pltpu.SemaphoreType.DMA,
    ],
)


def kernel(x_dict, edge_index_dict, edge_attr_dict, W_src, W_dst, att_src,
           att_dst, bias):
    x = x_dict
    ei = edge_index_dict
    e_in = ei.shape[1]
    e2 = e_in + N
    loop = jnp.arange(N, dtype=jnp.int32)
    src = jnp.concatenate([ei[0].astype(jnp.int32), loop,
                           jnp.zeros((E2P - e2,), jnp.int32)])
    dst = jnp.concatenate([ei[1].astype(jnp.int32), loop,
                           jnp.full((E2P - e2,), N, jnp.int32)])
    src2d = src.reshape(16 * NCH, CH)
    dst2d = dst.reshape(16 * NCH, CH)

    xp = jnp.pad(x, ((0, NP - N), (0, 0)))
    h4, a4 = _tc_call(xp, W_src, W_dst, att_src, att_dst)
    h4flat = h4.reshape(4 * NP, 128)
    a4t = jnp.transpose(a4, (1, 0))
    bias4 = bias.reshape(H, 2, 128).reshape(4, 128)

    out4 = _sc_call(h4flat, a4t, src2d, dst2d, bias4)
    out = out4[:, :N, :]
    out = jnp.transpose(out, (1, 0, 2)).reshape(N, HC)
    return out
